# baseline (speedup 1.0000x reference)
import functools
import numpy as np
import jax
import jax.numpy as jnp
from jax.experimental import pallas as pl
from jax.experimental.pallas import tpu as pltpu


def _fold_bn(gamma, beta, mean, var, eps=1e-5):
    scale = gamma / jnp.sqrt(var + eps)
    shift = beta - mean * scale
    return scale, shift


def _unit_kernel(x_ref, w1t_ref, b1_ref, wdw_ref, w2t_ref, b3_ref,
                 o_ref, *, h, w, ksize, nsub):
    hw = h * w
    col = jax.lax.broadcasted_iota(jnp.int32, (1, hw), 1) % w
    maskl = (col >= 1).astype(jnp.bfloat16)
    maskr = (col <= w - 2).astype(jnp.bfloat16)

    for j in range(nsub):
        x = x_ref[j]
        y = jnp.dot(w1t_ref[...], x,
                    preferred_element_type=jnp.float32)
        y = jnp.maximum(y + b1_ref[...], 0.0).astype(jnp.bfloat16)

        c = y.shape[0]
        z1 = jnp.zeros((c, 1), jnp.bfloat16)
        zw = jnp.zeros((c, w), jnp.bfloat16)
        t0 = jnp.concatenate([z1, y[:, :hw - 1]], 1) * maskl
        t1 = y
        t2 = jnp.concatenate([y[:, 1:], z1], 1) * maskr

        def urow(kh):
            return (t0 * wdw_ref[:, 3 * kh:3 * kh + 1]
                    + t1 * wdw_ref[:, 3 * kh + 1:3 * kh + 2]
                    + t2 * wdw_ref[:, 3 * kh + 2:3 * kh + 3])

        u0, u2 = urow(0), urow(2)
        z = (urow(1) + jnp.concatenate([zw, u0[:, :hw - w]], 1)
             + jnp.concatenate([u2[:, w:], zw], 1))

        out = jnp.dot(w2t_ref[...], z,
                      preferred_element_type=jnp.float32)
        out = jnp.maximum(out + b3_ref[...] + x.astype(jnp.float32), 0.0)
        o_ref[j] = out


def kernel(x, w1, wdw, w2,
           bn1_gamma, bn1_beta, bn1_mean, bn1_var,
           bn2_gamma, bn2_beta, bn2_mean, bn2_var,
           bn3_gamma, bn3_beta, bn3_mean, bn3_var):
    inp, oup, group = 256, 256, 4
    mid, ksize = 256, 3
    n, cin, h, w = x.shape
    assert cin == inp and oup == inp
    hw = h * w

    sc1, sh1 = _fold_bn(bn1_gamma, bn1_beta, bn1_mean, bn1_var)
    sc2, sh2 = _fold_bn(bn2_gamma, bn2_beta, bn2_mean, bn2_var)
    sc3, sh3 = _fold_bn(bn3_gamma, bn3_beta, bn3_mean, bn3_var)

    gc = mid // group
    cin_g, oc_g = inp // group, mid // group
    m1 = (np.arange(mid)[:, None] // oc_g == np.arange(inp)[None, :] // cin_g)
    w1t = (jnp.tile(w1[:, :, 0, 0], (1, group)) * m1
           * sc1[:, None]).astype(jnp.bfloat16)
    perm = np.arange(mid).reshape(gc, group).T.reshape(-1)
    iperm = np.argsort(perm)
    outputs = oup
    oc2 = outputs // group
    mm = iperm
    S = np.zeros((gc, mid), np.float32)
    S[mm % gc, np.arange(mid)] = 1.0
    M = (np.arange(outputs)[:, None] // oc2 == (mm // gc)[None, :])
    w2sq = w2[:, :, 0, 0]
    w2t_f32 = (w2sq @ S) * M * sc3[:, None]
    w2t = w2t_f32.astype(jnp.bfloat16)
    b1 = sh1[:, None]
    b3 = (w2t_f32 @ sh2 + sh3)[:, None]
    wdw_cm = (wdw[:, 0, :, :].reshape(mid, ksize * ksize)
              * sc2[:, None]).astype(jnp.bfloat16)

    xcm = x.reshape(n, inp, hw).astype(jnp.bfloat16)

    nsub = 1
    ncores = 2
    nper = n // (nsub * ncores)
    kern = functools.partial(_unit_kernel, h=h, w=w, ksize=ksize, nsub=nsub)
    out = pl.pallas_call(
        kern,
        out_shape=jax.ShapeDtypeStruct((n, oup, hw), jnp.float32),
        grid=(ncores, nper),
        in_specs=[
            pl.BlockSpec((nsub, inp, hw), lambda c, i: (c * nper + i, 0, 0)),
            pl.BlockSpec((mid, inp), lambda c, i: (0, 0)),
            pl.BlockSpec((mid, 1), lambda c, i: (0, 0)),
            pl.BlockSpec((mid, ksize * ksize), lambda c, i: (0, 0)),
            pl.BlockSpec((oup, mid), lambda c, i: (0, 0)),
            pl.BlockSpec((oup, 1), lambda c, i: (0, 0)),
        ],
        out_specs=pl.BlockSpec((nsub, oup, hw), lambda c, i: (c * nper + i, 0, 0)),
        compiler_params=pltpu.CompilerParams(
            dimension_semantics=("arbitrary", "arbitrary"),
            vmem_limit_bytes=int(32 << 20)),
    )(xcm, w1t, b1, wdw_cm, w2t, b3)
    return out.reshape(n, oup, h, w)
